# baseline (speedup 1.0000x reference)
import functools

import numpy as np
import jax
import jax.numpy as jnp
from jax import lax
from jax.experimental import pallas as pl
from jax.experimental.pallas import tpu as pltpu
from jax.experimental.pallas import tpu_sc as plsc

_NC = 2
_NS = 16
_NW = _NC * _NS

_CHUNK = 128


def _pe_np(seq_len: int, embed: int) -> np.ndarray:
    pos = np.arange(seq_len, dtype=np.float64)[:, None]
    denom = 10000.0 ** ((np.arange(embed) // 2).astype(np.float64) / embed)[None, :]
    ang = (pos / denom).astype(np.float32)
    pe = np.zeros((seq_len, embed), dtype=np.float32)
    pe[:, 0::2] = np.sin(ang[:, 0::2])
    pe[:, 1::2] = np.cos(ang[:, 1::2])
    return pe


@functools.partial(jax.jit, static_argnames=())
def kernel(input_x, table):
    batch, seq = input_x.shape
    vocab, embed = table.shape
    n_flat = batch * seq
    per_w = n_flat // _NW
    n_chunks = per_w // _CHUNK

    idx3 = input_x.reshape(_NW, n_chunks, _CHUNK).astype(jnp.int32)

    pe = _pe_np(seq, embed)
    pe2 = jnp.asarray(np.concatenate([pe, pe], axis=0))

    mesh = plsc.VectorSubcoreMesh(core_axis_name="c", subcore_axis_name="s")

    @functools.partial(
        pl.kernel,
        mesh=mesh,
        compiler_params=pltpu.CompilerParams(use_tc_tiling_on_sc=False),
        out_type=jax.ShapeDtypeStruct((n_flat, embed), jnp.float32),
        scratch_types=[
            pltpu.VMEM((n_chunks, _CHUNK), jnp.int32),
            pltpu.VMEM((2 * seq, embed), jnp.float32),
            pltpu.VMEM((_CHUNK, embed), jnp.float32),
            pltpu.SemaphoreType.DMA,
        ],
    )
    def _emb(idx_hbm, table_hbm, pe_hbm, out_hbm, idx_v, pe_v, rows_v, sem):
        wid = lax.axis_index("s") * _NC + lax.axis_index("c")
        pltpu.sync_copy(idx_hbm.at[wid], idx_v)
        pltpu.sync_copy(pe_hbm, pe_v)
        base = wid * per_w

        def chunk_body(i, carry):
            pltpu.async_copy(table_hbm.at[idx_v.at[i]], rows_v, sem).wait()
            off = (i * _CHUNK) % seq

            def row_body(r, c2):
                for e in range(embed // 16):
                    sl = pl.ds(e * 16, 16)
                    rows_v[r, sl] = rows_v[r, sl] + pe_v[off + r, sl]
                return c2

            lax.fori_loop(0, _CHUNK, row_body, 0)
            pltpu.sync_copy(rows_v, out_hbm.at[pl.ds(base + i * _CHUNK, _CHUNK)])
            return carry

        lax.fori_loop(0, n_chunks, chunk_body, 0)

    out_flat = _emb(idx3, table, pe2)
    return out_flat.reshape(batch, seq, embed)


# speedup vs baseline: 1.4559x; 1.4559x over previous
import functools

import numpy as np
import jax
import jax.numpy as jnp
from jax import lax
from jax.experimental import pallas as pl
from jax.experimental.pallas import tpu as pltpu
from jax.experimental.pallas import tpu_sc as plsc

_NC = 2
_NS = 16
_NW = _NC * _NS

_CHUNK = 128


def _pe_np(seq_len: int, embed: int) -> np.ndarray:
    pos = np.arange(seq_len, dtype=np.float64)[:, None]
    denom = 10000.0 ** ((np.arange(embed) // 2).astype(np.float64) / embed)[None, :]
    ang = (pos / denom).astype(np.float32)
    pe = np.zeros((seq_len, embed), dtype=np.float32)
    pe[:, 0::2] = np.sin(ang[:, 0::2])
    pe[:, 1::2] = np.cos(ang[:, 1::2])
    return pe


@jax.jit
def kernel(input_x, table):
    batch, seq = input_x.shape
    vocab, embed = table.shape
    n_flat = batch * seq
    per_w = n_flat // _NW
    n_chunks = per_w // _CHUNK
    n_vec = embed // 16

    idx3 = input_x.reshape(_NW, n_chunks, _CHUNK).astype(jnp.int32)

    pe = _pe_np(seq, embed)
    pe2 = jnp.asarray(np.concatenate([pe, pe], axis=0))

    mesh = plsc.VectorSubcoreMesh(core_axis_name="c", subcore_axis_name="s")

    @functools.partial(
        pl.kernel,
        mesh=mesh,
        compiler_params=pltpu.CompilerParams(use_tc_tiling_on_sc=False),
        out_type=jax.ShapeDtypeStruct((n_flat, embed), jnp.float32),
        scratch_types=[
            pltpu.VMEM((n_chunks, _CHUNK), jnp.int32),
            pltpu.VMEM((2 * seq, embed), jnp.float32),
            pltpu.VMEM((_CHUNK, embed), jnp.float32),
            pltpu.VMEM((_CHUNK, embed), jnp.float32),
            pltpu.SemaphoreType.DMA,
            pltpu.SemaphoreType.DMA,
            pltpu.SemaphoreType.DMA,
            pltpu.SemaphoreType.DMA,
        ],
    )
    def _emb(idx_hbm, table_hbm, pe_hbm, out_hbm, idx_v, pe_v,
             rows0, rows1, sg0, sg1, so0, so1):
        rb = (rows0, rows1)
        sg = (sg0, sg1)
        so = (so0, so1)
        wid = lax.axis_index("s") * _NC + lax.axis_index("c")
        pltpu.sync_copy(idx_hbm.at[wid], idx_v)
        pltpu.sync_copy(pe_hbm, pe_v)
        base = wid * per_w

        pltpu.async_copy(table_hbm.at[idx_v.at[0]], rb[0], sg[0])

        def out_slice(i):
            return out_hbm.at[pl.ds(base + i * _CHUNK, _CHUNK)]

        def step(i, b):
            b2 = 1 - b

            @pl.when(i > 0)
            def _():
                pltpu.make_async_copy(rb[b2], out_slice(i - 1), so[b2]).wait()

            @pl.when(i + 1 < n_chunks)
            def _():
                pltpu.async_copy(table_hbm.at[idx_v.at[i + 1]], rb[b2], sg[b2])

            pltpu.make_async_copy(table_hbm.at[idx_v.at[i]], rb[b], sg[b]).wait()

            off = (i * _CHUNK) % seq

            @plsc.parallel_loop(0, _CHUNK, unroll=8)
            def _(r):
                for e in range(n_vec):
                    sl = pl.ds(e * 16, 16)
                    rb[b][r, sl] = rb[b][r, sl] + pe_v[off + r, sl]

            pltpu.async_copy(rb[b], out_slice(i), so[b])

        def loop_body(t, carry):
            for b in range(2):
                step(2 * t + b, b)
            return carry

        lax.fori_loop(0, n_chunks // 2, loop_body, 0)

        pltpu.make_async_copy(rb[1], out_slice(n_chunks - 1), so[1]).wait()

    out_flat = _emb(idx3, table, pe2)
    return out_flat.reshape(batch, seq, embed)


# speedup vs baseline: 1.5388x; 1.0570x over previous
import functools

import numpy as np
import jax
import jax.numpy as jnp
from jax import lax
from jax.experimental import pallas as pl
from jax.experimental.pallas import tpu as pltpu
from jax.experimental.pallas import tpu_sc as plsc

_NC = 2
_NS = 16
_NW = _NC * _NS

_CHUNK = 128


def _pe_np(seq_len: int, embed: int) -> np.ndarray:
    pos = np.arange(seq_len, dtype=np.float64)[:, None]
    denom = 10000.0 ** ((np.arange(embed) // 2).astype(np.float64) / embed)[None, :]
    ang = (pos / denom).astype(np.float32)
    pe = np.zeros((seq_len, embed), dtype=np.float32)
    pe[:, 0::2] = np.sin(ang[:, 0::2])
    pe[:, 1::2] = np.cos(ang[:, 1::2])
    return pe


@jax.jit
def kernel(input_x, table):
    batch, seq = input_x.shape
    vocab, embed = table.shape
    n_flat = batch * seq
    per_w = n_flat // _NW
    n_chunks = per_w // _CHUNK
    n_vec = embed // 16

    idx3 = input_x.reshape(_NW, n_chunks, _CHUNK).astype(jnp.int32)

    pe = _pe_np(seq, embed)
    pe2 = jnp.asarray(np.concatenate([pe, pe], axis=0))

    mesh = plsc.VectorSubcoreMesh(core_axis_name="c", subcore_axis_name="s")

    NB = 8
    LOOKAHEAD = 5

    @functools.partial(
        pl.kernel,
        mesh=mesh,
        compiler_params=pltpu.CompilerParams(use_tc_tiling_on_sc=False),
        out_type=jax.ShapeDtypeStruct((n_flat, embed), jnp.float32),
        scratch_types=[
            pltpu.VMEM((n_chunks, _CHUNK), jnp.int32),
            pltpu.VMEM((2 * seq, embed), jnp.float32),
        ]
        + [pltpu.VMEM((_CHUNK, embed), jnp.float32)] * NB
        + [pltpu.SemaphoreType.DMA] * (2 * NB),
    )
    def _emb(idx_hbm, table_hbm, pe_hbm, out_hbm, idx_v, pe_v, *bufs):
        rb = bufs[:NB]
        sg = bufs[NB:2 * NB]
        so = bufs[2 * NB:]
        wid = lax.axis_index("s") * _NC + lax.axis_index("c")
        pltpu.sync_copy(idx_hbm.at[wid], idx_v)
        pltpu.sync_copy(pe_hbm, pe_v)
        base = wid * per_w

        def out_slice(i):
            return out_hbm.at[pl.ds(base + i * _CHUNK, _CHUNK)]

        for j in range(LOOKAHEAD):
            pltpu.async_copy(table_hbm.at[idx_v.at[j]], rb[j], sg[j])

        def step(i, b):
            bg = (b + LOOKAHEAD) % NB

            @pl.when(i >= NB - LOOKAHEAD)
            def _():
                pltpu.make_async_copy(
                    rb[bg], out_slice(i - (NB - LOOKAHEAD)), so[bg]).wait()

            @pl.when(i + LOOKAHEAD < n_chunks)
            def _():
                pltpu.async_copy(
                    table_hbm.at[idx_v.at[i + LOOKAHEAD]], rb[bg], sg[bg])

            pltpu.make_async_copy(table_hbm.at[idx_v.at[i]], rb[b], sg[b]).wait()

            off = (i * _CHUNK) % seq

            @plsc.parallel_loop(0, _CHUNK, unroll=8)
            def _(r):
                for e in range(n_vec):
                    sl = pl.ds(e * 16, 16)
                    rb[b][r, sl] = rb[b][r, sl] + pe_v[off + r, sl]

            pltpu.async_copy(rb[b], out_slice(i), so[b])

        def loop_body(t, carry):
            for b in range(NB):
                step(NB * t + b, b)
            return carry

        lax.fori_loop(0, n_chunks // NB, loop_body, 0)

        for j in range(NB - LOOKAHEAD):
            i = n_chunks - (NB - LOOKAHEAD) + j
            pltpu.make_async_copy(rb[i % NB], out_slice(i), so[i % NB]).wait()

    out_flat = _emb(idx3, table, pe2)
    return out_flat.reshape(batch, seq, embed)
